# baseline (speedup 1.0000x reference)
import functools

import jax
import jax.numpy as jnp
from jax import lax
from jax.experimental import pallas as pl
from jax.experimental.pallas import tpu as pltpu
from jax.experimental.pallas import tpu_sc as plsc

G, N, E, D, L = 2, 10000, 320000, 128, 2
H = D // 2
NS = 16
NC = 2
CH = 64
BLK = 8
EPT = 20480
NCHUNK = EPT // CH
NPAIR = NCHUNK // 16
EPAD = EPT * NS
ACC_ROWS = 10240
ZPT = ACC_ROWS // NS
OCH = 64
NFC = N // OCH
ZB = 8
ROW_U = 8


def _sc_edge_body(h2, srcr, dst2d, t16, out,
                  acc, zbuf, sblk, dblk,
                  rows0, rows1, outv0, outv1, obuf, tv,
                  gsem0, gsem1, ssem0, ssem1, isem):
    cid = lax.axis_index("c")
    sid = lax.axis_index("s")
    rowss = (rows0, rows1)
    outvs = (outv0, outv1)
    gsems = (gsem0, gsem1)
    ssems = (ssem0, ssem1)

    zeros16 = jnp.zeros((16,), jnp.float32)
    for r in range(ZB):
        for v in range(D // 16):
            zbuf[r, pl.ds(v * 16, 16)] = zeros16

    zbase = sid * ZPT

    def zloop(k, carry):
        pltpu.sync_copy(zbuf, acc.at[pl.ds(zbase + k * ZB, ZB)])
        return carry

    lax.fori_loop(0, ZPT // ZB, zloop, 0)
    pltpu.sync_copy(t16, tv)
    plsc.subcore_barrier()

    tvec = tv[...]
    bias = cid * N
    ebase = sid * EPT
    cbase = sid * NCHUNK

    def blk_refs(oct_idx):
        soff = ebase + oct_idx * BLK * CH
        return (srcr.at[pl.ds(soff, BLK * CH)],
                dst2d.at[pl.ds(cbase + oct_idx * BLK, BLK)])

    def fetch_blk(oct_idx, bb):
        s, d2 = blk_refs(oct_idx)
        pltpu.async_copy(s, sblk.at[bb], isem)
        pltpu.async_copy(d2, dblk.at[bb], isem)

    def wait_bias_blk(oct_idx, bb):
        s, d2 = blk_refs(oct_idx)
        pltpu.make_async_copy(s, sblk.at[bb], isem).wait()
        pltpu.make_async_copy(d2, dblk.at[bb], isem).wait()
        for v in range(BLK * CH // 16):
            sblk[bb, pl.ds(v * 16, 16)] = sblk[bb, pl.ds(v * 16, 16)] + bias

    def src_idx(p):
        pw = p % 16
        return sblk.at[pw // BLK, pl.ds((pw % BLK) * CH, CH)]

    def dst_idx(p):
        pw = p % 16
        return dblk.at[pw // BLK, (pw % BLK)]

    def compute(rows, outv):
        @plsc.parallel_loop(0, CH, step=1, unroll=ROW_U)
        def crow(r):
            for v in range(H // 16):
                y = rows[r, pl.ds(v * 16, 16)]
                m = jnp.maximum(y, 0.0)
                e = jnp.exp(m * tvec)
                outv[r, pl.ds(v * 16, 16)] = m * e
                outv[r, pl.ds(H + v * 16, 16)] = e

    s0, d0 = blk_refs(0)
    pltpu.sync_copy(s0, sblk.at[0])
    pltpu.sync_copy(d0, dblk.at[0])
    for v in range(BLK * CH // 16):
        sblk[0, pl.ds(v * 16, 16)] = sblk[0, pl.ds(v * 16, 16)] + bias
    pltpu.async_copy(h2.at[src_idx(0)], rows0, gsem0)

    def pair(ko, carry):
        for p in range(16):
            k = 16 * ko + p
            b, bn = p % 2, (p + 1) % 2

            if p == 2:
                fetch_blk(2 * ko + 1, 1)
            if p == 10:
                @pl.when(ko + 1 < NPAIR)
                def _fetch0():
                    fetch_blk(2 * (ko + 1), 0)
            if p == 6:
                wait_bias_blk(2 * ko + 1, 1)
            if p == 14:
                @pl.when(ko + 1 < NPAIR)
                def _bias0():
                    wait_bias_blk(2 * (ko + 1), 0)

            @pl.when(k + 1 < NCHUNK)
            def _prefetch():
                pltpu.async_copy(h2.at[src_idx(p + 1)], rowss[bn],
                                 gsems[bn])

            pltpu.make_async_copy(h2.at[src_idx(p)], rowss[b],
                                  gsems[b]).wait()

            @pl.when(k >= 2)
            def _drain():
                pltpu.make_async_copy(outvs[b], acc.at[dst_idx(p - 2)],
                                      ssems[b]).wait()

            compute(rowss[b], outvs[b])
            pltpu.async_copy(outvs[b], acc.at[dst_idx(p)], ssems[b],
                             add=True)
        return carry

    lax.fori_loop(0, NPAIR, pair, 0)
    pltpu.make_async_copy(outv0, acc.at[dst_idx(14)], ssem0).wait()
    pltpu.make_async_copy(outv1, acc.at[dst_idx(15)], ssem1).wait()
    plsc.subcore_barrier()

    def finchunk(rb, nrows):
        pltpu.sync_copy(acc.at[pl.ds(rb, nrows)], outv0.at[pl.ds(0, nrows)])

        def frow(r, c2):
            for v in range(H // 16):
                num = outv0[r, pl.ds(v * 16, 16)]
                den = outv0[r, pl.ds(H + v * 16, 16)]
                obuf[r, pl.ds(v * 16, 16)] = num / (den + 1e-16) + 1e-7
            return c2

        lax.fori_loop(0, nrows, frow, 0)
        pltpu.sync_copy(obuf.at[pl.ds(0, nrows)],
                        out.at[pl.ds(cid * N + rb, nrows)])

    def fin(j, carry):
        c = sid + NS * j

        @pl.when(c < NFC)
        def _():
            finchunk(c * OCH, OCH)

        return carry

    lax.fori_loop(0, (NFC + NS - 1) // NS, fin, 0)

    @pl.when(sid == 0)
    def _tail():
        finchunk(NFC * OCH, N - NFC * OCH)


_sc_edge = functools.partial(
    pl.kernel,
    out_type=jax.ShapeDtypeStruct((2 * N, H), jnp.float32),
    mesh=plsc.VectorSubcoreMesh(core_axis_name="c", subcore_axis_name="s"),
    compiler_params=pltpu.CompilerParams(use_tc_tiling_on_sc=False),
    scratch_types=[
        pltpu.VMEM_SHARED((ACC_ROWS, D), jnp.float32),
        pltpu.VMEM((ZB, D), jnp.float32),
        pltpu.VMEM((2, BLK * CH), jnp.int32),
        pltpu.VMEM((2, BLK, CH), jnp.int32),
        pltpu.VMEM((CH, H), jnp.float32),
        pltpu.VMEM((CH, H), jnp.float32),
        pltpu.VMEM((CH, D), jnp.float32),
        pltpu.VMEM((CH, D), jnp.float32),
        pltpu.VMEM((OCH, H), jnp.float32),
        pltpu.VMEM((16,), jnp.float32),
        pltpu.SemaphoreType.DMA,
        pltpu.SemaphoreType.DMA,
        pltpu.SemaphoreType.DMA,
        pltpu.SemaphoreType.DMA,
        pltpu.SemaphoreType.DMA,
    ],
)(_sc_edge_body)


BA = 400
BC = 400


def _ln_body(x_ref, g_ref, b_ref, o_ref):
    x = x_ref[...]
    m = jnp.mean(x, axis=-1, keepdims=True)
    v = jnp.mean((x - m) ** 2, axis=-1, keepdims=True)
    h = (x - m) * lax.rsqrt(v + 1e-5) * g_ref[...] + b_ref[...]
    o_ref[0] = h[:, :H]
    o_ref[1] = h[:, H:]


def _ln(x, g, b):
    return pl.pallas_call(
        _ln_body,
        grid=(N // BA,),
        in_specs=[
            pl.BlockSpec((BA, D), lambda i: (i, 0)),
            pl.BlockSpec((1, D), lambda i: (0, 0)),
            pl.BlockSpec((1, D), lambda i: (0, 0)),
        ],
        out_specs=pl.BlockSpec((2, BA, H), lambda i: (0, i, 0)),
        out_shape=jax.ShapeDtypeStruct((2, N, H), jnp.float32),
    )(x, g.reshape(1, D), b.reshape(1, D))


def _mlp_body(apply_relu, x_ref, a_ref, g_ref, b_ref, w1_ref, b1_ref,
              mg_ref, mb_ref, w2_ref, b2_ref, o_ref):
    x = x_ref[...]
    m = jnp.mean(x, axis=-1, keepdims=True)
    v = jnp.mean((x - m) ** 2, axis=-1, keepdims=True)
    h = (x - m) * lax.rsqrt(v + 1e-5) * g_ref[...] + b_ref[...]
    agg = jnp.concatenate([a_ref[0], a_ref[1]], axis=1)
    out = agg + h
    hm = jnp.dot(out, w1_ref[...], preferred_element_type=jnp.float32)
    hm = hm + b1_ref[...]
    mm = jnp.mean(hm, axis=-1, keepdims=True)
    mv = jnp.mean((hm - mm) ** 2, axis=-1, keepdims=True)
    hm = (hm - mm) * lax.rsqrt(mv + 1e-5) * mg_ref[...] + mb_ref[...]
    hm = jnp.maximum(hm, 0.0)
    y = jnp.dot(hm, w2_ref[...], preferred_element_type=jnp.float32)
    y = y + b2_ref[...] + x
    if apply_relu:
        y = jnp.maximum(y, 0.0)
    o_ref[...] = y


def _mlp(x, agg, g, b, w1, b1, mg, mb, w2, b2, apply_relu):
    return pl.pallas_call(
        functools.partial(_mlp_body, apply_relu),
        grid=(N // BC,),
        in_specs=[
            pl.BlockSpec((BC, D), lambda i: (i, 0)),
            pl.BlockSpec((2, BC, H), lambda i: (0, i, 0)),
            pl.BlockSpec((1, D), lambda i: (0, 0)),
            pl.BlockSpec((1, D), lambda i: (0, 0)),
            pl.BlockSpec((D, 2 * D), lambda i: (0, 0)),
            pl.BlockSpec((1, 2 * D), lambda i: (0, 0)),
            pl.BlockSpec((1, 2 * D), lambda i: (0, 0)),
            pl.BlockSpec((1, 2 * D), lambda i: (0, 0)),
            pl.BlockSpec((2 * D, D), lambda i: (0, 0)),
            pl.BlockSpec((1, D), lambda i: (0, 0)),
        ],
        out_specs=pl.BlockSpec((BC, D), lambda i: (i, 0)),
        out_shape=jax.ShapeDtypeStruct((N, D), jnp.float32),
    )(x, agg, g.reshape(1, D), b.reshape(1, D), w1, b1.reshape(1, 2 * D),
      mg.reshape(1, 2 * D), mb.reshape(1, 2 * D), w2, b2.reshape(1, D))


def kernel(input_nodes, input_edges, params):
    pad = EPAD - E
    outs = []
    for gi in range(G):
        src = input_edges[gi, 0].astype(jnp.int32)
        dst = input_edges[gi, 1].astype(jnp.int32)
        src_p = jnp.concatenate([src, jnp.zeros((pad,), jnp.int32)])
        dst_p = jnp.concatenate([dst, jnp.full((pad,), N, jnp.int32)])
        dst2d = dst_p.reshape(EPAD // CH, CH)
        x = input_nodes[gi]
        for l in range(L):
            g, b, t, w1, b1, mg, mb, w2, b2 = params[l]
            h2 = _ln(x, g, b).reshape(2 * N, H)
            t16 = jnp.full((16,), t, jnp.float32)
            agg = _sc_edge(h2, src_p, dst2d, t16).reshape(2, N, H)
            x = _mlp(x, agg, g, b, w1, b1, mg, mb, w2, b2,
                     apply_relu=(l < L - 1))
        outs.append(x)
    return jnp.stack(outs, axis=0)


# speedup vs baseline: 1.2395x; 1.2395x over previous
import functools

import jax
import jax.numpy as jnp
from jax import lax
from jax.experimental import pallas as pl
from jax.experimental.pallas import tpu as pltpu
from jax.experimental.pallas import tpu_sc as plsc

G, N, E, D, L = 2, 10000, 320000, 128, 2
H = D // 2
NS = 16
NC = 2
CH = 64
BLK = 4
EPT = 20224
NCHUNK = EPT // CH
NQUAD = NCHUNK // BLK
EPAD = EPT * NS
ACC_ROWS = 10240
ZPT = ACC_ROWS // NS
OCH = 64
NFC = N // OCH
ZB = 8
ROW_U = 8


def _sc_edge_body(h2, srcr, dst2d, t16, out,
                  acc, zbuf, sblk, dblk,
                  rows0, rows1, outv0, outv1, obuf, tv,
                  gsem0, gsem1, ssem0, ssem1, isem):
    cid = lax.axis_index("c")
    sid = lax.axis_index("s")
    rowss = (rows0, rows1)
    outvs = (outv0, outv1)
    gsems = (gsem0, gsem1)
    ssems = (ssem0, ssem1)

    zeros16 = jnp.zeros((16,), jnp.float32)
    for r in range(ZB):
        for v in range(D // 16):
            zbuf[r, pl.ds(v * 16, 16)] = zeros16

    zbase = sid * ZPT

    def zloop(k, carry):
        pltpu.sync_copy(zbuf, acc.at[pl.ds(zbase + k * ZB, ZB)])
        return carry

    lax.fori_loop(0, ZPT // ZB, zloop, 0)
    pltpu.sync_copy(t16, tv)
    plsc.subcore_barrier()

    tvec = tv[...]
    bias = cid * N
    ebase = sid * EPT
    cbase = sid * NCHUNK
    BW = BLK * CH

    def blk_refs(qi, par):
        soff = ebase + qi * BW
        return (srcr.at[pl.ds(soff, BW)], sblk.at[pl.ds(par * BW, BW)],
                dst2d.at[pl.ds(cbase + qi * BLK, BLK)],
                dblk.at[pl.ds(par * BLK, BLK)])

    def fetch_blk(qi, par):
        s, sv, d2, dv = blk_refs(qi, par)
        pltpu.async_copy(s, sv, isem)
        pltpu.async_copy(d2, dv, isem)

    def wait_bias_blk(qi, par):
        s, sv, d2, dv = blk_refs(qi, par)
        pltpu.make_async_copy(s, sv, isem).wait()
        pltpu.make_async_copy(d2, dv, isem).wait()
        base = par * BW
        for v in range(BW // 16):
            o = base + v * 16
            sblk[pl.ds(o, 16)] = sblk[pl.ds(o, 16)] + bias

    def src_idx(par, j):
        return sblk.at[pl.ds(par * BW + j * CH, CH)]

    def dst_idx(par, j):
        return dblk.at[par * BLK + j]

    def compute(rows, outv):
        @plsc.parallel_loop(0, CH, step=1, unroll=ROW_U)
        def crow(r):
            for v in range(H // 16):
                y = rows[r, pl.ds(v * 16, 16)]
                m = jnp.maximum(y, 0.0)
                e = jnp.exp(m * tvec)
                outv[r, pl.ds(v * 16, 16)] = m * e
                outv[r, pl.ds(H + v * 16, 16)] = e

    s, sv, d2, dv = blk_refs(0, 0)
    pltpu.sync_copy(s, sv)
    pltpu.sync_copy(d2, dv)
    for v in range(BW // 16):
        sblk[pl.ds(v * 16, 16)] = sblk[pl.ds(v * 16, 16)] + bias
    pltpu.async_copy(h2.at[src_idx(0, 0)], rows0, gsem0)

    def quad(ko, carry):
        par = ko % 3
        parn = (ko + 1) % 3

        @pl.when(ko + 1 < NQUAD)
        def _fetch():
            fetch_blk(ko + 1, parn)

        for j in range(4):
            k = 4 * ko + j
            b, bn = j % 2, (j + 1) % 2

            if j == 2:
                @pl.when(ko + 1 < NQUAD)
                def _wb():
                    wait_bias_blk(ko + 1, parn)

            @pl.when(k + 1 < NCHUNK)
            def _prefetch():
                nref = (src_idx(par, j + 1) if j < 3 else src_idx(parn, 0))
                pltpu.async_copy(h2.at[nref], rowss[bn], gsems[bn])

            pltpu.make_async_copy(h2.at[src_idx(par, j)], rowss[b],
                                  gsems[b]).wait()

            @pl.when(k >= 2)
            def _drain():
                pltpu.make_async_copy(outvs[b], acc.at[dst_idx(0, 0)],
                                      ssems[b]).wait()

            compute(rowss[b], outvs[b])
            pltpu.async_copy(outvs[b], acc.at[dst_idx(par, j)], ssems[b],
                             add=True)
        return carry

    lax.fori_loop(0, NQUAD, quad, 0)
    pltpu.make_async_copy(outv0, acc.at[dst_idx(0, 0)], ssem0).wait()
    pltpu.make_async_copy(outv1, acc.at[dst_idx(0, 1)], ssem1).wait()
    plsc.subcore_barrier()

    def finchunk(rb, nrows):
        pltpu.sync_copy(acc.at[pl.ds(rb, nrows)], outv0.at[pl.ds(0, nrows)])

        def frow(r, c2):
            for v in range(H // 16):
                num = outv0[r, pl.ds(v * 16, 16)]
                den = outv0[r, pl.ds(H + v * 16, 16)]
                obuf[r, pl.ds(v * 16, 16)] = num / (den + 1e-16) + 1e-7
            return c2

        lax.fori_loop(0, nrows, frow, 0)
        pltpu.sync_copy(obuf.at[pl.ds(0, nrows)],
                        out.at[pl.ds(cid * N + rb, nrows)])

    def fin(j, carry):
        c = sid + NS * j

        @pl.when(c < NFC)
        def _():
            finchunk(c * OCH, OCH)

        return carry

    lax.fori_loop(0, (NFC + NS - 1) // NS, fin, 0)

    @pl.when(sid == 0)
    def _tail():
        finchunk(NFC * OCH, N - NFC * OCH)


_sc_edge = functools.partial(
    pl.kernel,
    out_type=jax.ShapeDtypeStruct((2 * N, H), jnp.float32),
    mesh=plsc.VectorSubcoreMesh(core_axis_name="c", subcore_axis_name="s"),
    compiler_params=pltpu.CompilerParams(use_tc_tiling_on_sc=False),
    scratch_types=[
        pltpu.VMEM_SHARED((ACC_ROWS, D), jnp.float32),
        pltpu.VMEM((ZB, D), jnp.float32),
        pltpu.VMEM((3 * BLK * CH,), jnp.int32),
        pltpu.VMEM((3 * BLK, CH), jnp.int32),
        pltpu.VMEM((CH, H), jnp.float32),
        pltpu.VMEM((CH, H), jnp.float32),
        pltpu.VMEM((CH, D), jnp.float32),
        pltpu.VMEM((CH, D), jnp.float32),
        pltpu.VMEM((OCH, H), jnp.float32),
        pltpu.VMEM((16,), jnp.float32),
        pltpu.SemaphoreType.DMA,
        pltpu.SemaphoreType.DMA,
        pltpu.SemaphoreType.DMA,
        pltpu.SemaphoreType.DMA,
        pltpu.SemaphoreType.DMA,
    ],
)(_sc_edge_body)


BA = 400
BC = 400


def _ln_body(x_ref, g_ref, b_ref, o_ref):
    x = x_ref[...]
    m = jnp.mean(x, axis=-1, keepdims=True)
    v = jnp.mean((x - m) ** 2, axis=-1, keepdims=True)
    h = (x - m) * lax.rsqrt(v + 1e-5) * g_ref[...] + b_ref[...]
    o_ref[0] = h[:, :H]
    o_ref[1] = h[:, H:]


def _ln(x, g, b):
    return pl.pallas_call(
        _ln_body,
        grid=(N // BA,),
        in_specs=[
            pl.BlockSpec((BA, D), lambda i: (i, 0)),
            pl.BlockSpec((1, D), lambda i: (0, 0)),
            pl.BlockSpec((1, D), lambda i: (0, 0)),
        ],
        out_specs=pl.BlockSpec((2, BA, H), lambda i: (0, i, 0)),
        out_shape=jax.ShapeDtypeStruct((2, N, H), jnp.float32),
    )(x, g.reshape(1, D), b.reshape(1, D))


def _mlp_body(apply_relu, x_ref, a_ref, g_ref, b_ref, w1_ref, b1_ref,
              mg_ref, mb_ref, w2_ref, b2_ref, o_ref):
    x = x_ref[...]
    m = jnp.mean(x, axis=-1, keepdims=True)
    v = jnp.mean((x - m) ** 2, axis=-1, keepdims=True)
    h = (x - m) * lax.rsqrt(v + 1e-5) * g_ref[...] + b_ref[...]
    agg = jnp.concatenate([a_ref[0], a_ref[1]], axis=1)
    out = agg + h
    hm = jnp.dot(out, w1_ref[...], preferred_element_type=jnp.float32)
    hm = hm + b1_ref[...]
    mm = jnp.mean(hm, axis=-1, keepdims=True)
    mv = jnp.mean((hm - mm) ** 2, axis=-1, keepdims=True)
    hm = (hm - mm) * lax.rsqrt(mv + 1e-5) * mg_ref[...] + mb_ref[...]
    hm = jnp.maximum(hm, 0.0)
    y = jnp.dot(hm, w2_ref[...], preferred_element_type=jnp.float32)
    y = y + b2_ref[...] + x
    if apply_relu:
        y = jnp.maximum(y, 0.0)
    o_ref[...] = y


def _mlp(x, agg, g, b, w1, b1, mg, mb, w2, b2, apply_relu):
    return pl.pallas_call(
        functools.partial(_mlp_body, apply_relu),
        grid=(N // BC,),
        in_specs=[
            pl.BlockSpec((BC, D), lambda i: (i, 0)),
            pl.BlockSpec((2, BC, H), lambda i: (0, i, 0)),
            pl.BlockSpec((1, D), lambda i: (0, 0)),
            pl.BlockSpec((1, D), lambda i: (0, 0)),
            pl.BlockSpec((D, 2 * D), lambda i: (0, 0)),
            pl.BlockSpec((1, 2 * D), lambda i: (0, 0)),
            pl.BlockSpec((1, 2 * D), lambda i: (0, 0)),
            pl.BlockSpec((1, 2 * D), lambda i: (0, 0)),
            pl.BlockSpec((2 * D, D), lambda i: (0, 0)),
            pl.BlockSpec((1, D), lambda i: (0, 0)),
        ],
        out_specs=pl.BlockSpec((BC, D), lambda i: (i, 0)),
        out_shape=jax.ShapeDtypeStruct((N, D), jnp.float32),
    )(x, agg, g.reshape(1, D), b.reshape(1, D), w1, b1.reshape(1, 2 * D),
      mg.reshape(1, 2 * D), mb.reshape(1, 2 * D), w2, b2.reshape(1, D))


def kernel(input_nodes, input_edges, params):
    pad = EPAD - E
    outs = []
    for gi in range(G):
        src = input_edges[gi, 0].astype(jnp.int32)
        dst = input_edges[gi, 1].astype(jnp.int32)
        src_p = jnp.concatenate([src, jnp.zeros((pad,), jnp.int32)])
        dst_p = jnp.concatenate([dst, jnp.full((pad,), N, jnp.int32)])
        dst2d = dst_p.reshape(EPAD // CH, CH)
        x = input_nodes[gi]
        for l in range(L):
            g, b, t, w1, b1, mg, mb, w2, b2 = params[l]
            h2 = _ln(x, g, b).reshape(2 * N, H)
            t16 = jnp.full((16,), t, jnp.float32)
            agg = _sc_edge(h2, src_p, dst2d, t16).reshape(2, N, H)
            x = _mlp(x, agg, g, b, w1, b1, mg, mb, w2, b2,
                     apply_relu=(l < L - 1))
        outs.append(x)
    return jnp.stack(outs, axis=0)


# speedup vs baseline: 1.2581x; 1.0150x over previous
import functools

import jax
import jax.numpy as jnp
from jax import lax
from jax.experimental import pallas as pl
from jax.experimental.pallas import tpu as pltpu
from jax.experimental.pallas import tpu_sc as plsc

G, N, E, D, L = 2, 10000, 320000, 128, 2
H = D // 2
NS = 16
NC = 2
CH = 64
BLK = 4
EPT = 20224
NCHUNK = EPT // CH
NQUAD = NCHUNK // BLK
EPAD = EPT * NS
ACC_ROWS = 10240
ZPT = ACC_ROWS // NS
OCH = 64
NFC = N // OCH
ZB = 32
ROW_U = 8


def _sc_edge_body(h2, srcr, dst2d, t16, out,
                  acc, zbuf, sblk, dblk,
                  rows0, rows1, outv0, outv1, obuf, tv,
                  gsem0, gsem1, ssem0, ssem1, isem):
    cid = lax.axis_index("c")
    sid = lax.axis_index("s")
    rowss = (rows0, rows1)
    outvs = (outv0, outv1)
    gsems = (gsem0, gsem1)
    ssems = (ssem0, ssem1)

    zeros16 = jnp.zeros((16,), jnp.float32)
    for r in range(ZB):
        for v in range(D // 16):
            zbuf[r, pl.ds(v * 16, 16)] = zeros16

    zbase = sid * ZPT

    def zloop(k, carry):
        pltpu.async_copy(zbuf, acc.at[pl.ds(zbase + k * ZB, ZB)], isem)
        return carry

    lax.fori_loop(0, ZPT // ZB, zloop, 0)
    pltpu.sync_copy(t16, tv)

    def zdrain(k, carry):
        pltpu.make_async_copy(zbuf, acc.at[pl.ds(zbase, ZB)], isem).wait()
        return carry

    lax.fori_loop(0, ZPT // ZB, zdrain, 0)
    plsc.subcore_barrier()

    tvec = tv[...]
    bias = cid * N
    ebase = sid * EPT
    cbase = sid * NCHUNK
    BW = BLK * CH

    def blk_refs(qi, par):
        soff = ebase + qi * BW
        return (srcr.at[pl.ds(soff, BW)], sblk.at[pl.ds(par * BW, BW)],
                dst2d.at[pl.ds(cbase + qi * BLK, BLK)],
                dblk.at[pl.ds(par * BLK, BLK)])

    def fetch_blk(qi, par):
        s, sv, d2, dv = blk_refs(qi, par)
        pltpu.async_copy(s, sv, isem)
        pltpu.async_copy(d2, dv, isem)

    def wait_bias_blk(qi, par):
        s, sv, d2, dv = blk_refs(qi, par)
        pltpu.make_async_copy(s, sv, isem).wait()
        pltpu.make_async_copy(d2, dv, isem).wait()
        base = par * BW
        for v in range(BW // 16):
            o = base + v * 16
            sblk[pl.ds(o, 16)] = sblk[pl.ds(o, 16)] + bias

    def src_idx(par, j):
        return sblk.at[pl.ds(par * BW + j * CH, CH)]

    def dst_idx(par, j):
        return dblk.at[par * BLK + j]

    def compute(rows, outv):
        @plsc.parallel_loop(0, CH, step=1, unroll=ROW_U)
        def crow(r):
            for v in range(H // 16):
                y = rows[r, pl.ds(v * 16, 16)]
                m = jnp.maximum(y, 0.0)
                e = jnp.exp(m * tvec)
                outv[r, pl.ds(v * 16, 16)] = m * e
                outv[r, pl.ds(H + v * 16, 16)] = e

    s, sv, d2, dv = blk_refs(0, 0)
    pltpu.sync_copy(s, sv)
    pltpu.sync_copy(d2, dv)
    for v in range(BW // 16):
        sblk[pl.ds(v * 16, 16)] = sblk[pl.ds(v * 16, 16)] + bias
    pltpu.async_copy(h2.at[src_idx(0, 0)], rows0, gsem0)

    def quad(ko, carry):
        par = ko % 3
        parn = (ko + 1) % 3

        @pl.when(ko + 1 < NQUAD)
        def _fetch():
            fetch_blk(ko + 1, parn)

        for j in range(4):
            k = 4 * ko + j
            b, bn = j % 2, (j + 1) % 2

            if j == 2:
                @pl.when(ko + 1 < NQUAD)
                def _wb():
                    wait_bias_blk(ko + 1, parn)

            @pl.when(k + 1 < NCHUNK)
            def _prefetch():
                nref = (src_idx(par, j + 1) if j < 3 else src_idx(parn, 0))
                pltpu.async_copy(h2.at[nref], rowss[bn], gsems[bn])

            pltpu.make_async_copy(h2.at[src_idx(par, j)], rowss[b],
                                  gsems[b]).wait()

            @pl.when(k >= 2)
            def _drain():
                pltpu.make_async_copy(outvs[b], acc.at[dst_idx(0, 0)],
                                      ssems[b]).wait()

            compute(rowss[b], outvs[b])
            pltpu.async_copy(outvs[b], acc.at[dst_idx(par, j)], ssems[b],
                             add=True)
        return carry

    lax.fori_loop(0, NQUAD, quad, 0)
    pltpu.make_async_copy(outv0, acc.at[dst_idx(0, 0)], ssem0).wait()
    pltpu.make_async_copy(outv1, acc.at[dst_idx(0, 1)], ssem1).wait()
    plsc.subcore_barrier()

    def finchunk(rb, nrows):
        pltpu.sync_copy(acc.at[pl.ds(rb, nrows)], outv0.at[pl.ds(0, nrows)])

        def frow(r, c2):
            for v in range(H // 16):
                num = outv0[r, pl.ds(v * 16, 16)]
                den = outv0[r, pl.ds(H + v * 16, 16)]
                obuf[r, pl.ds(v * 16, 16)] = num / (den + 1e-16) + 1e-7
            return c2

        lax.fori_loop(0, nrows, frow, 0)
        pltpu.sync_copy(obuf.at[pl.ds(0, nrows)],
                        out.at[pl.ds(cid * N + rb, nrows)])

    def fin(j, carry):
        c = sid + NS * j

        @pl.when(c < NFC)
        def _():
            finchunk(c * OCH, OCH)

        return carry

    lax.fori_loop(0, (NFC + NS - 1) // NS, fin, 0)

    @pl.when(sid == 0)
    def _tail():
        finchunk(NFC * OCH, N - NFC * OCH)


_sc_edge = functools.partial(
    pl.kernel,
    out_type=jax.ShapeDtypeStruct((2 * N, H), jnp.float32),
    mesh=plsc.VectorSubcoreMesh(core_axis_name="c", subcore_axis_name="s"),
    compiler_params=pltpu.CompilerParams(use_tc_tiling_on_sc=False),
    scratch_types=[
        pltpu.VMEM_SHARED((ACC_ROWS, D), jnp.float32),
        pltpu.VMEM((ZB, D), jnp.float32),
        pltpu.VMEM((3 * BLK * CH,), jnp.int32),
        pltpu.VMEM((3 * BLK, CH), jnp.int32),
        pltpu.VMEM((CH, H), jnp.float32),
        pltpu.VMEM((CH, H), jnp.float32),
        pltpu.VMEM((CH, D), jnp.float32),
        pltpu.VMEM((CH, D), jnp.float32),
        pltpu.VMEM((OCH, H), jnp.float32),
        pltpu.VMEM((16,), jnp.float32),
        pltpu.SemaphoreType.DMA,
        pltpu.SemaphoreType.DMA,
        pltpu.SemaphoreType.DMA,
        pltpu.SemaphoreType.DMA,
        pltpu.SemaphoreType.DMA,
    ],
)(_sc_edge_body)


BA = 400
BC = 400


def _ln_body(x_ref, g_ref, b_ref, o_ref):
    x = x_ref[...]
    m = jnp.mean(x, axis=-1, keepdims=True)
    v = jnp.mean((x - m) ** 2, axis=-1, keepdims=True)
    h = (x - m) * lax.rsqrt(v + 1e-5) * g_ref[...] + b_ref[...]
    o_ref[0] = h[:, :H]
    o_ref[1] = h[:, H:]


def _ln(x, g, b):
    return pl.pallas_call(
        _ln_body,
        grid=(N // BA,),
        in_specs=[
            pl.BlockSpec((BA, D), lambda i: (i, 0)),
            pl.BlockSpec((1, D), lambda i: (0, 0)),
            pl.BlockSpec((1, D), lambda i: (0, 0)),
        ],
        out_specs=pl.BlockSpec((2, BA, H), lambda i: (0, i, 0)),
        out_shape=jax.ShapeDtypeStruct((2, N, H), jnp.float32),
    )(x, g.reshape(1, D), b.reshape(1, D))


def _mlp_body(apply_relu, x_ref, a_ref, g_ref, b_ref, w1_ref, b1_ref,
              mg_ref, mb_ref, w2_ref, b2_ref, o_ref):
    x = x_ref[...]
    m = jnp.mean(x, axis=-1, keepdims=True)
    v = jnp.mean((x - m) ** 2, axis=-1, keepdims=True)
    h = (x - m) * lax.rsqrt(v + 1e-5) * g_ref[...] + b_ref[...]
    agg = jnp.concatenate([a_ref[0], a_ref[1]], axis=1)
    out = agg + h
    hm = jnp.dot(out, w1_ref[...], preferred_element_type=jnp.float32)
    hm = hm + b1_ref[...]
    mm = jnp.mean(hm, axis=-1, keepdims=True)
    mv = jnp.mean((hm - mm) ** 2, axis=-1, keepdims=True)
    hm = (hm - mm) * lax.rsqrt(mv + 1e-5) * mg_ref[...] + mb_ref[...]
    hm = jnp.maximum(hm, 0.0)
    y = jnp.dot(hm, w2_ref[...], preferred_element_type=jnp.float32)
    y = y + b2_ref[...] + x
    if apply_relu:
        y = jnp.maximum(y, 0.0)
    o_ref[...] = y


def _mlp(x, agg, g, b, w1, b1, mg, mb, w2, b2, apply_relu):
    return pl.pallas_call(
        functools.partial(_mlp_body, apply_relu),
        grid=(N // BC,),
        in_specs=[
            pl.BlockSpec((BC, D), lambda i: (i, 0)),
            pl.BlockSpec((2, BC, H), lambda i: (0, i, 0)),
            pl.BlockSpec((1, D), lambda i: (0, 0)),
            pl.BlockSpec((1, D), lambda i: (0, 0)),
            pl.BlockSpec((D, 2 * D), lambda i: (0, 0)),
            pl.BlockSpec((1, 2 * D), lambda i: (0, 0)),
            pl.BlockSpec((1, 2 * D), lambda i: (0, 0)),
            pl.BlockSpec((1, 2 * D), lambda i: (0, 0)),
            pl.BlockSpec((2 * D, D), lambda i: (0, 0)),
            pl.BlockSpec((1, D), lambda i: (0, 0)),
        ],
        out_specs=pl.BlockSpec((BC, D), lambda i: (i, 0)),
        out_shape=jax.ShapeDtypeStruct((N, D), jnp.float32),
    )(x, agg, g.reshape(1, D), b.reshape(1, D), w1, b1.reshape(1, 2 * D),
      mg.reshape(1, 2 * D), mb.reshape(1, 2 * D), w2, b2.reshape(1, D))


def kernel(input_nodes, input_edges, params):
    pad = EPAD - E
    outs = []
    for gi in range(G):
        src = input_edges[gi, 0].astype(jnp.int32)
        dst = input_edges[gi, 1].astype(jnp.int32)
        src_p = jnp.concatenate([src, jnp.zeros((pad,), jnp.int32)])
        dst_p = jnp.concatenate([dst, jnp.full((pad,), N, jnp.int32)])
        dst2d = dst_p.reshape(EPAD // CH, CH)
        x = input_nodes[gi]
        for l in range(L):
            g, b, t, w1, b1, mg, mb, w2, b2 = params[l]
            h2 = _ln(x, g, b).reshape(2 * N, H)
            t16 = jnp.full((16,), t, jnp.float32)
            agg = _sc_edge(h2, src_p, dst2d, t16).reshape(2, N, H)
            x = _mlp(x, agg, g, b, w1, b1, mg, mb, w2, b2,
                     apply_relu=(l < L - 1))
        outs.append(x)
    return jnp.stack(outs, axis=0)
